# baseline (speedup 1.0000x reference)
import functools

import jax
import jax.numpy as jnp
from jax import lax
from jax.experimental import pallas as pl
from jax.experimental.pallas import tpu as pltpu
from jax.experimental.pallas import tpu_sc as plsc

V = 32000
PAD = 0
SMOOTHING = 0.1
CONF = 1.0 - SMOOTHING
EPS = SMOOTHING / (V - 2)

N_ROWS = 4096

BR = 128
BC = V

NSC = 1024
NW = 32
RPW = NSC // NW
W16 = V // 16
W128 = V // 128
ROW_OFF = NSC // BR


def _tc_kernel(t_ref, x_ref, out_ref):
    x = x_ref[...]
    t = t_ref[0, 0, :]
    m = jnp.max(x, axis=1, keepdims=True)
    s = jnp.sum(jnp.exp(x - m), axis=1, keepdims=True)
    lane = jax.lax.broadcasted_iota(jnp.int32, x.shape, 1)
    wt = jnp.where(lane == t[:, None], CONF, EPS)
    w = jnp.sum(wt * x, axis=1, keepdims=True)
    w = w - EPS * x[:, 0:1]
    L = m + jnp.log(s)
    out_ref[...] = jnp.where(t[:, None] == PAD, 0.0, L - w)


def _dyn_gather(row, idx):
    dn = lax.GatherDimensionNumbers(
        offset_dims=(), collapsed_slice_dims=(0,), start_index_map=(0,))
    return lax.gather(row, idx[:, None], dn, (1,),
                      mode=lax.GatherScatterMode.PROMISE_IN_BOUNDS)

def _sc_kernel(p_hbm, p128_hbm, t_hbm,
               om_hbm, os_hbm, osp_hbm, opt_hbm, op0_hbm,
               buf0, buf1, tloc, macc, sacc, spacc, ptacc, p0acc,
               win, sem0, sem1, semg):
    wid = lax.axis_index("s") * 2 + lax.axis_index("c")
    base = wid * RPW

    pltpu.sync_copy(t_hbm.at[pl.ds(base, RPW)], tloc)

    bufs = (buf0, buf1)
    sems = (sem0, sem1)
    copies = [None, None]
    copies[0] = pltpu.async_copy(p_hbm.at[base], buf0, sem0)

    neg_inf = jnp.full((16,), -jnp.inf, jnp.float32)
    zeros = jnp.zeros((16,), jnp.float32)

    for j in range(RPW):
        buf = bufs[j % 2]
        copies[j % 2].wait()
        if j + 1 < RPW:
            copies[(j + 1) % 2] = pltpu.async_copy(
                p_hbm.at[base + j + 1], bufs[(j + 1) % 2], sems[(j + 1) % 2])

        def _max_body(i, m):
            return jnp.maximum(m, buf[pl.ds(i * 16, 16)])

        m = lax.fori_loop(0, W16, _max_body, neg_inf)

        def _sum_body(i, carry):
            s, sp = carry
            v = buf[pl.ds(i * 16, 16)]
            return s + jnp.exp(v - m), sp + v

        s, sp = lax.fori_loop(0, W16, _sum_body, (zeros, zeros))

        macc[pl.ds(j * 16, 16)] = m
        sacc[pl.ds(j * 16, 16)] = s
        spacc[pl.ds(j * 16, 16)] = sp

    lane16 = lax.broadcasted_iota(jnp.int32, (16,), 0)
    zeros_i = jnp.zeros((16,), jnp.int32)
    for g in range(RPW // 16):
        tv = tloc[pl.ds(g * 16, 16)]
        rows_vec = (base + g * 16 + lane16) * W128
        idxw = rows_vec + jnp.right_shift(tv, 7)
        tlow = jnp.bitwise_and(tv, 15)
        thi = jnp.bitwise_and(jnp.right_shift(tv, 4), 7)
        pltpu.async_copy(p128_hbm.at[idxw], win, semg).wait()
        ptv = zeros
        for j in range(16):
            for k in range(8):
                vreg = win[j, pl.ds(k * 16, 16)]
                gj = _dyn_gather(vreg, tlow)
                ptv = jnp.where((lane16 == j) & (thi == k), gj, ptv)
        ptacc[pl.ds(g * 16, 16)] = ptv
        pltpu.async_copy(p128_hbm.at[rows_vec], win, semg).wait()
        p0v = zeros
        for j in range(16):
            vreg = win[j, pl.ds(0, 16)]
            gj = _dyn_gather(vreg, zeros_i)
            p0v = jnp.where(lane16 == j, gj, p0v)
        p0acc[pl.ds(g * 16, 16)] = p0v

    pltpu.sync_copy(macc, om_hbm.at[pl.ds(base * 16, RPW * 16)])
    pltpu.sync_copy(sacc, os_hbm.at[pl.ds(base * 16, RPW * 16)])
    pltpu.sync_copy(spacc, osp_hbm.at[pl.ds(base * 16, RPW * 16)])
    pltpu.sync_copy(ptacc, opt_hbm.at[pl.ds(base, RPW)])
    pltpu.sync_copy(p0acc, op0_hbm.at[pl.ds(base, RPW)])


_sc_call = functools.partial(
    pl.kernel,
    mesh=plsc.VectorSubcoreMesh(core_axis_name="c", subcore_axis_name="s"),
    out_type=[
        jax.ShapeDtypeStruct((NSC * 16,), jnp.float32),
        jax.ShapeDtypeStruct((NSC * 16,), jnp.float32),
        jax.ShapeDtypeStruct((NSC * 16,), jnp.float32),
        jax.ShapeDtypeStruct((NSC,), jnp.float32),
        jax.ShapeDtypeStruct((NSC,), jnp.float32),
    ],
    scratch_types=[
        pltpu.VMEM((V,), jnp.float32),
        pltpu.VMEM((V,), jnp.float32),
        pltpu.VMEM((RPW,), jnp.int32),
        pltpu.VMEM((RPW * 16,), jnp.float32),
        pltpu.VMEM((RPW * 16,), jnp.float32),
        pltpu.VMEM((RPW * 16,), jnp.float32),
        pltpu.VMEM((RPW,), jnp.float32),
        pltpu.VMEM((RPW,), jnp.float32),
        pltpu.VMEM((16, 128), jnp.float32),
        pltpu.SemaphoreType.DMA,
        pltpu.SemaphoreType.DMA,
        pltpu.SemaphoreType.DMA,
    ],
)(_sc_kernel)


def kernel(pred, target):
    n = N_ROWS
    p = pred.reshape(n, V)
    t = target.reshape(-1).astype(jnp.int32)

    p128 = p.reshape(n * W128, 128)
    om, os_, osp, pt, p0 = _sc_call(p, p128, t)

    nr = (n - NSC) // BR
    t3 = t[NSC:].reshape(nr, 1, BR)
    tc_loss = pl.pallas_call(
        _tc_kernel,
        grid=(nr,),
        in_specs=[
            pl.BlockSpec((1, 1, BR), lambda r: (r, 0, 0)),
            pl.BlockSpec((BR, BC), lambda r: (r + ROW_OFF, 0)),
        ],
        out_specs=pl.BlockSpec((BR, 1), lambda r: (r, 0)),
        out_shape=jax.ShapeDtypeStruct((n - NSC, 1), jnp.float32),
        compiler_params=pltpu.CompilerParams(
            dimension_semantics=("parallel",)),
    )(t3, p)

    m_l = om.reshape(NSC, 16)
    s_l = os_.reshape(NSC, 16)
    sp = osp.reshape(NSC, 16).sum(axis=1)
    M = m_l.max(axis=1)
    S = jnp.sum(s_l * jnp.exp(m_l - M[:, None]), axis=1)
    L = M + jnp.log(S)
    w = EPS * (sp - p0) + (CONF - EPS) * pt
    t_sc = t[:NSC]
    sc_loss = jnp.where(t_sc == PAD, 0.0, L - w)

    return (jnp.sum(sc_loss) + jnp.sum(tc_loss)) / n


# speedup vs baseline: 1.6291x; 1.6291x over previous
import functools

import jax
import jax.numpy as jnp
from jax import lax
from jax.experimental import pallas as pl
from jax.experimental.pallas import tpu as pltpu
from jax.experimental.pallas import tpu_sc as plsc

V = 32000
PAD = 0
SMOOTHING = 0.1
CONF = 1.0 - SMOOTHING
EPS = SMOOTHING / (V - 2)

N_ROWS = 4096

BR = 128
BC = V

NSC = 1024
NW = 32
RPW = NSC // NW
W16 = V // 16
W128 = V // 128
ROW_OFF = NSC // BR


def _tc_kernel(t_ref, x_ref, out_ref):
    x = x_ref[...]
    t = t_ref[0, 0, :]
    m = jnp.max(x, axis=1, keepdims=True)
    s = jnp.sum(jnp.exp(x - m), axis=1, keepdims=True)
    lane = jax.lax.broadcasted_iota(jnp.int32, x.shape, 1)
    wt = jnp.where(lane == t[:, None], CONF, EPS)
    w = jnp.sum(wt * x, axis=1, keepdims=True)
    w = w - EPS * x[:, 0:1]
    L = m + jnp.log(s)
    out_ref[...] = jnp.where(t[:, None] == PAD, 0.0, L - w)


def _dyn_gather(row, idx):
    dn = lax.GatherDimensionNumbers(
        offset_dims=(), collapsed_slice_dims=(0,), start_index_map=(0,))
    return lax.gather(row, idx[:, None], dn, (1,),
                      mode=lax.GatherScatterMode.PROMISE_IN_BOUNDS)

def _sc_kernel(p_hbm, p128_hbm, t_hbm,
               om_hbm, os_hbm, osp_hbm, opt_hbm, op0_hbm,
               buf0, buf1, tloc, macc, sacc, spacc, ptacc, p0acc,
               win, sem0, sem1, semg):
    wid = lax.axis_index("s") * 2 + lax.axis_index("c")
    base = wid * RPW

    pltpu.sync_copy(t_hbm.at[pl.ds(base, RPW)], tloc)

    bufs = (buf0, buf1)
    sems = (sem0, sem1)
    copies = [None, None]
    copies[0] = pltpu.async_copy(p_hbm.at[base], buf0, sem0)

    neg_inf = jnp.full((16,), -jnp.inf, jnp.float32)
    zeros = jnp.zeros((16,), jnp.float32)

    for j in range(RPW):
        buf = bufs[j % 2]
        copies[j % 2].wait()
        if j + 1 < RPW:
            copies[(j + 1) % 2] = pltpu.async_copy(
                p_hbm.at[base + j + 1], bufs[(j + 1) % 2], sems[(j + 1) % 2])

        def _max_body(i, carry):
            b = i * 128
            out = []
            for k in range(4):
                mk = carry[k]
                mk = jnp.maximum(mk, buf[pl.ds(b + k * 16, 16)])
                mk = jnp.maximum(mk, buf[pl.ds(b + (k + 4) * 16, 16)])
                out.append(mk)
            return tuple(out)

        m4 = lax.fori_loop(0, W16 // 8, _max_body,
                           (neg_inf, neg_inf, neg_inf, neg_inf))
        m = jnp.maximum(jnp.maximum(m4[0], m4[1]),
                        jnp.maximum(m4[2], m4[3]))

        def _sum_body(i, carry):
            b = i * 128
            ss = list(carry[0])
            pp = list(carry[1])
            for k in range(8):
                v = buf[pl.ds(b + k * 16, 16)]
                ss[k % 4] = ss[k % 4] + jnp.exp(v - m)
                pp[k % 4] = pp[k % 4] + v
            return tuple(ss), tuple(pp)

        (s4, sp4) = lax.fori_loop(
            0, W16 // 8, _sum_body,
            ((zeros,) * 4, (zeros,) * 4))
        s = (s4[0] + s4[1]) + (s4[2] + s4[3])
        sp = (sp4[0] + sp4[1]) + (sp4[2] + sp4[3])

        macc[pl.ds(j * 16, 16)] = m
        sacc[pl.ds(j * 16, 16)] = s
        spacc[pl.ds(j * 16, 16)] = sp

    lane16 = lax.broadcasted_iota(jnp.int32, (16,), 0)
    zeros_i = jnp.zeros((16,), jnp.int32)
    for g in range(RPW // 16):
        tv = tloc[pl.ds(g * 16, 16)]
        rows_vec = (base + g * 16 + lane16) * W128
        idxw = rows_vec + jnp.right_shift(tv, 7)
        tlow = jnp.bitwise_and(tv, 15)
        thi = jnp.bitwise_and(jnp.right_shift(tv, 4), 7)
        pltpu.async_copy(p128_hbm.at[idxw], win, semg).wait()
        ptv = zeros
        for j in range(16):
            for k in range(8):
                vreg = win[j, pl.ds(k * 16, 16)]
                gj = _dyn_gather(vreg, tlow)
                ptv = jnp.where((lane16 == j) & (thi == k), gj, ptv)
        ptacc[pl.ds(g * 16, 16)] = ptv
        pltpu.async_copy(p128_hbm.at[rows_vec], win, semg).wait()
        p0v = zeros
        for j in range(16):
            vreg = win[j, pl.ds(0, 16)]
            gj = _dyn_gather(vreg, zeros_i)
            p0v = jnp.where(lane16 == j, gj, p0v)
        p0acc[pl.ds(g * 16, 16)] = p0v

    pltpu.sync_copy(macc, om_hbm.at[pl.ds(base * 16, RPW * 16)])
    pltpu.sync_copy(sacc, os_hbm.at[pl.ds(base * 16, RPW * 16)])
    pltpu.sync_copy(spacc, osp_hbm.at[pl.ds(base * 16, RPW * 16)])
    pltpu.sync_copy(ptacc, opt_hbm.at[pl.ds(base, RPW)])
    pltpu.sync_copy(p0acc, op0_hbm.at[pl.ds(base, RPW)])


_sc_call = functools.partial(
    pl.kernel,
    mesh=plsc.VectorSubcoreMesh(core_axis_name="c", subcore_axis_name="s"),
    out_type=[
        jax.ShapeDtypeStruct((NSC * 16,), jnp.float32),
        jax.ShapeDtypeStruct((NSC * 16,), jnp.float32),
        jax.ShapeDtypeStruct((NSC * 16,), jnp.float32),
        jax.ShapeDtypeStruct((NSC,), jnp.float32),
        jax.ShapeDtypeStruct((NSC,), jnp.float32),
    ],
    scratch_types=[
        pltpu.VMEM((V,), jnp.float32),
        pltpu.VMEM((V,), jnp.float32),
        pltpu.VMEM((RPW,), jnp.int32),
        pltpu.VMEM((RPW * 16,), jnp.float32),
        pltpu.VMEM((RPW * 16,), jnp.float32),
        pltpu.VMEM((RPW * 16,), jnp.float32),
        pltpu.VMEM((RPW,), jnp.float32),
        pltpu.VMEM((RPW,), jnp.float32),
        pltpu.VMEM((16, 128), jnp.float32),
        pltpu.SemaphoreType.DMA,
        pltpu.SemaphoreType.DMA,
        pltpu.SemaphoreType.DMA,
    ],
)(_sc_kernel)


def kernel(pred, target):
    n = N_ROWS
    p = pred.reshape(n, V)
    t = target.reshape(-1).astype(jnp.int32)

    p128 = p.reshape(n * W128, 128)
    om, os_, osp, pt, p0 = _sc_call(p, p128, t)

    nr = (n - NSC) // BR
    t3 = t[NSC:].reshape(nr, 1, BR)
    tc_loss = pl.pallas_call(
        _tc_kernel,
        grid=(nr,),
        in_specs=[
            pl.BlockSpec((1, 1, BR), lambda r: (r, 0, 0)),
            pl.BlockSpec((BR, BC), lambda r: (r + ROW_OFF, 0)),
        ],
        out_specs=pl.BlockSpec((BR, 1), lambda r: (r, 0)),
        out_shape=jax.ShapeDtypeStruct((n - NSC, 1), jnp.float32),
        compiler_params=pltpu.CompilerParams(
            dimension_semantics=("parallel",)),
    )(t3, p)

    m_l = om.reshape(NSC, 16)
    s_l = os_.reshape(NSC, 16)
    sp = osp.reshape(NSC, 16).sum(axis=1)
    M = m_l.max(axis=1)
    S = jnp.sum(s_l * jnp.exp(m_l - M[:, None]), axis=1)
    L = M + jnp.log(S)
    w = EPS * (sp - p0) + (CONF - EPS) * pt
    t_sc = t[:NSC]
    sc_loss = jnp.where(t_sc == PAD, 0.0, L - w)

    return (jnp.sum(sc_loss) + jnp.sum(tc_loss)) / n


# speedup vs baseline: 4.6234x; 2.8381x over previous
import functools

import jax
import jax.numpy as jnp
from jax import lax
from jax.experimental import pallas as pl
from jax.experimental.pallas import tpu as pltpu
from jax.experimental.pallas import tpu_sc as plsc

V = 32000
PAD = 0
SMOOTHING = 0.1
CONF = 1.0 - SMOOTHING
EPS = SMOOTHING / (V - 2)

N_ROWS = 4096

BR = 128
BC = V

NSC = 1024
NW = 32
RPW = NSC // NW
W16 = V // 16
ROW_OFF = NSC // BR


def _tc_kernel(t_ref, x_ref, out_ref):
    x = x_ref[...]
    t = t_ref[0, 0, :]
    m = jnp.max(x, axis=1, keepdims=True)
    s = jnp.sum(jnp.exp(x - m), axis=1, keepdims=True)
    lane = jax.lax.broadcasted_iota(jnp.int32, x.shape, 1)
    wt = jnp.where(lane == t[:, None], CONF, EPS)
    w = jnp.sum(wt * x, axis=1, keepdims=True)
    w = w - EPS * x[:, 0:1]
    L = m + jnp.log(s)
    out_ref[...] = jnp.where(t[:, None] == PAD, 0.0, L - w)


def _dyn_gather(row, idx):
    dn = lax.GatherDimensionNumbers(
        offset_dims=(), collapsed_slice_dims=(0,), start_index_map=(0,))
    return lax.gather(row, idx[:, None], dn, (1,),
                      mode=lax.GatherScatterMode.PROMISE_IN_BOUNDS)


def _sc_kernel(p_hbm, t_hbm, om_hbm, os_hbm, ow_hbm,
               buf0, buf1, tloc, macc, sacc, wacc, sem0, sem1):
    wid = lax.axis_index("s") * 2 + lax.axis_index("c")
    base = wid * RPW

    pltpu.sync_copy(t_hbm.at[pl.ds(base, RPW)], tloc)

    bufs = (buf0, buf1)
    sems = (sem0, sem1)
    copies = [None, None]
    copies[0] = pltpu.async_copy(p_hbm.at[base], buf0, sem0)

    neg_inf = jnp.full((16,), -jnp.inf, jnp.float32)
    zeros = jnp.zeros((16,), jnp.float32)
    zeros_i = jnp.zeros((16,), jnp.int32)
    lane16 = lax.broadcasted_iota(jnp.int32, (16,), 0)

    for j in range(RPW):
        buf = bufs[j % 2]
        copies[j % 2].wait()
        if j + 1 < RPW:
            copies[(j + 1) % 2] = pltpu.async_copy(
                p_hbm.at[base + j + 1], bufs[(j + 1) % 2], sems[(j + 1) % 2])

        tv = tloc[pl.ds((j // 16) * 16, 16)]
        tj = _dyn_gather(tv, jnp.full((16,), j % 16, jnp.int32))

        def _max_body(i, carry):
            b = i * 128
            out = []
            for k in range(4):
                mk = carry[k]
                mk = jnp.maximum(mk, buf[pl.ds(b + k * 16, 16)])
                mk = jnp.maximum(mk, buf[pl.ds(b + (k + 4) * 16, 16)])
                out.append(mk)
            return tuple(out)

        m4 = lax.fori_loop(0, W16 // 8, _max_body,
                           (neg_inf, neg_inf, neg_inf, neg_inf))
        m = jnp.maximum(jnp.maximum(m4[0], m4[1]),
                        jnp.maximum(m4[2], m4[3]))

        def _sum_body(i, carry):
            b = i * 128
            ss = list(carry[0])
            ww = list(carry[1])
            for k in range(8):
                v = buf[pl.ds(b + k * 16, 16)]
                col = lane16 + (b + k * 16)
                ss[k % 4] = ss[k % 4] + jnp.exp(v - m)
                wt = jnp.where(col == tj, CONF, EPS)
                ww[k % 4] = ww[k % 4] + wt * v
            return tuple(ss), tuple(ww)

        (s4, w4) = lax.fori_loop(
            0, W16 // 8, _sum_body, ((zeros,) * 4, (zeros,) * 4))
        s = (s4[0] + s4[1]) + (s4[2] + s4[3])
        w = (w4[0] + w4[1]) + (w4[2] + w4[3])

        v0 = buf[pl.ds(0, 16)]
        p0 = _dyn_gather(v0, zeros_i)
        w = w - jnp.where(lane16 == 0, EPS * p0, 0.0)

        macc[pl.ds(j * 16, 16)] = m
        sacc[pl.ds(j * 16, 16)] = s
        wacc[pl.ds(j * 16, 16)] = w

    pltpu.sync_copy(macc, om_hbm.at[pl.ds(base * 16, RPW * 16)])
    pltpu.sync_copy(sacc, os_hbm.at[pl.ds(base * 16, RPW * 16)])
    pltpu.sync_copy(wacc, ow_hbm.at[pl.ds(base * 16, RPW * 16)])


_sc_call = functools.partial(
    pl.kernel,
    mesh=plsc.VectorSubcoreMesh(core_axis_name="c", subcore_axis_name="s"),
    out_type=[
        jax.ShapeDtypeStruct((NSC * 16,), jnp.float32),
        jax.ShapeDtypeStruct((NSC * 16,), jnp.float32),
        jax.ShapeDtypeStruct((NSC * 16,), jnp.float32),
    ],
    scratch_types=[
        pltpu.VMEM((V,), jnp.float32),
        pltpu.VMEM((V,), jnp.float32),
        pltpu.VMEM((RPW,), jnp.int32),
        pltpu.VMEM((RPW * 16,), jnp.float32),
        pltpu.VMEM((RPW * 16,), jnp.float32),
        pltpu.VMEM((RPW * 16,), jnp.float32),
        pltpu.SemaphoreType.DMA,
        pltpu.SemaphoreType.DMA,
    ],
)(_sc_kernel)


def kernel(pred, target):
    n = N_ROWS
    p = pred.reshape(n, V)
    t = target.reshape(-1).astype(jnp.int32)

    om, os_, ow = _sc_call(p, t)

    nr = (n - NSC) // BR
    t3 = t[NSC:].reshape(nr, 1, BR)
    tc_loss = pl.pallas_call(
        _tc_kernel,
        grid=(nr,),
        in_specs=[
            pl.BlockSpec((1, 1, BR), lambda r: (r, 0, 0)),
            pl.BlockSpec((BR, BC), lambda r: (r + ROW_OFF, 0)),
        ],
        out_specs=pl.BlockSpec((BR, 1), lambda r: (r, 0)),
        out_shape=jax.ShapeDtypeStruct((n - NSC, 1), jnp.float32),
        compiler_params=pltpu.CompilerParams(
            dimension_semantics=("parallel",)),
    )(t3, p)

    m_l = om.reshape(NSC, 16)
    s_l = os_.reshape(NSC, 16)
    w = ow.reshape(NSC, 16).sum(axis=1)
    M = m_l.max(axis=1)
    S = jnp.sum(s_l * jnp.exp(m_l - M[:, None]), axis=1)
    L = M + jnp.log(S)
    t_sc = t[:NSC]
    sc_loss = jnp.where(t_sc == PAD, 0.0, L - w)

    return (jnp.sum(sc_loss) + jnp.sum(tc_loss)) / n
